# baseline (speedup 1.0000x reference)
import functools

import jax
import jax.numpy as jnp
from jax import lax
from jax.experimental import pallas as pl
from jax.experimental.pallas import tpu as pltpu
from jax.experimental.pallas import tpu_sc as plsc

MAXLEN = 200
EMBED = 32
BATCH = 4096

NC = 2
NS = 16
NW = NC * NS
LANES = 16

B_W = BATCH // NW
CH_B = 16
CH_P = 40
NB = B_W // CH_B
NP = MAXLEN // CH_P
TOK = CH_B * CH_P
PITCH = CH_P * EMBED

_mesh = plsc.VectorSubcoreMesh(core_axis_name="c", subcore_axis_name="s")


@functools.partial(
    pl.kernel,
    out_type=jax.ShapeDtypeStruct((MAXLEN, EMBED, BATCH), jnp.float32),
    mesh=_mesh,
    compiler_params=pltpu.CompilerParams(
        use_tc_tiling_on_sc=False, needs_layout_passes=False
    ),
    scratch_types=[
        pltpu.VMEM((CH_B, CH_P), jnp.int32),
        pltpu.VMEM((TOK, EMBED), jnp.float32),
        pltpu.VMEM((CH_P, EMBED, CH_B), jnp.float32),
        pltpu.VMEM((MAXLEN, EMBED), jnp.float32),
        pltpu.SemaphoreType.DMA,
    ],
)
def _embed_sc(x_hbm, tok_hbm, pos_hbm, out_hbm, idx_v, gbuf, obuf, pos_v, sem):
    wid = lax.axis_index("s") * NC + lax.axis_index("c")
    pltpu.sync_copy(pos_hbm, pos_v)
    stride_v = lax.iota(jnp.int32, LANES) * CH_P

    def chunk_body(c, _):
        b0 = wid * B_W + CH_B * (c // NP)
        p0 = CH_P * (c % NP)
        pltpu.sync_copy(x_hbm.at[pl.ds(b0, CH_B), pl.ds(p0, CH_P)], idx_v)
        copies = []
        for bl in range(CH_B):
            copies.append(
                pltpu.async_copy(
                    tok_hbm.at[idx_v.at[bl]],
                    gbuf.at[pl.ds(bl * CH_P, CH_P)],
                    sem,
                )
            )
        for cp in copies:
            cp.wait()

        def pos_body(p, _):
            rows = stride_v + p
            psplat_row = jnp.full((LANES,), p0 + p, jnp.int32)
            for e in range(EMBED):
                esplat = jnp.full((LANES,), e, jnp.int32)
                src = plsc.load_gather(gbuf, [rows, esplat])
                psplat = plsc.load_gather(pos_v, [psplat_row, esplat])
                obuf[p, e, :] = src + psplat
            return 0

        lax.fori_loop(0, CH_P, pos_body, 0)
        pltpu.sync_copy(
            obuf, out_hbm.at[pl.ds(p0, CH_P), :, pl.ds(b0, CH_B)]
        )
        return 0

    lax.fori_loop(0, NB * NP, chunk_body, 0)


def kernel(x, token_table, pos_table):
    out = _embed_sc(x.astype(jnp.int32), token_table, pos_table)
    return jnp.transpose(out, (2, 0, 1))


# speedup vs baseline: 1.2294x; 1.2294x over previous
import functools

import jax
import jax.numpy as jnp
from jax import lax
from jax.experimental import pallas as pl
from jax.experimental.pallas import tpu as pltpu
from jax.experimental.pallas import tpu_sc as plsc

MAXLEN = 200
EMBED = 32
BATCH = 4096

NC = 2
NS = 16
NW = NC * NS
LANES = 16

TOTAL = BATCH * MAXLEN
PER_W = TOTAL // NW
CHUNK = 1024
GROUP = 128
NGROUP = CHUNK // GROUP
NCHUNK = PER_W // CHUNK
HALVES = EMBED // LANES

_mesh = plsc.VectorSubcoreMesh(core_axis_name="c", subcore_axis_name="s")


@functools.partial(
    pl.kernel,
    out_type=jax.ShapeDtypeStruct((NW * NCHUNK, CHUNK, EMBED), jnp.float32),
    mesh=_mesh,
    compiler_params=pltpu.CompilerParams(use_tc_tiling_on_sc=False),
    scratch_types=[
        pltpu.VMEM((NGROUP, GROUP), jnp.int32),
        pltpu.VMEM((CHUNK, EMBED), jnp.float32),
        pltpu.VMEM((MAXLEN, EMBED), jnp.float32),
        pltpu.SemaphoreType.DMA,
    ],
)
def _embed_sc(x_hbm, tok_hbm, pos_hbm, out_hbm, idx_v, rows_v, pos_v, sem):
    wid = lax.axis_index("s") * NC + lax.axis_index("c")
    pltpu.sync_copy(pos_hbm, pos_v)

    def chunk_body(c, _):
        j = wid * NCHUNK + c
        pltpu.sync_copy(x_hbm.at[j], idx_v)
        copies = []
        for g in range(NGROUP):
            copies.append(
                pltpu.async_copy(
                    tok_hbm.at[idx_v.at[g]],
                    rows_v.at[pl.ds(g * GROUP, GROUP)],
                    sem,
                )
            )
        for cp in copies:
            cp.wait()

        p0 = lax.rem(j * CHUNK, MAXLEN)

        def row_body(r, p):
            p = jnp.where(p >= MAXLEN, p - MAXLEN, p)
            for h in range(HALVES):
                sl = pl.ds(h * LANES, LANES)
                plsc.addupdate(rows_v.at[r, sl], pos_v[p, sl])
            return p + 1

        lax.fori_loop(0, CHUNK, row_body, p0, unroll=4)
        pltpu.sync_copy(rows_v, out_hbm.at[j])
        return 0

    lax.fori_loop(0, NCHUNK, chunk_body, 0)


def kernel(x, token_table, pos_table):
    xr = x.astype(jnp.int32).reshape(NW * NCHUNK, NGROUP, GROUP)
    out = _embed_sc(xr, token_table, pos_table)
    return out.reshape(BATCH, MAXLEN, EMBED)


# speedup vs baseline: 1.2773x; 1.0390x over previous
import functools

import jax
import jax.numpy as jnp
from jax import lax
from jax.experimental import pallas as pl
from jax.experimental.pallas import tpu as pltpu
from jax.experimental.pallas import tpu_sc as plsc

MAXLEN = 200
EMBED = 32
BATCH = 4096

NC = 2
NS = 16
NW = NC * NS
LANES = 16

TOTAL = BATCH * MAXLEN
PER_W = TOTAL // NW
CHUNK = 800
GROUP = 80
NGROUP = CHUNK // GROUP
NCHUNK = PER_W // CHUNK
HALVES = EMBED // LANES

_mesh = plsc.VectorSubcoreMesh(core_axis_name="c", subcore_axis_name="s")


@functools.partial(
    pl.kernel,
    out_type=jax.ShapeDtypeStruct((NW * NCHUNK, CHUNK, EMBED), jnp.float32),
    mesh=_mesh,
    compiler_params=pltpu.CompilerParams(use_tc_tiling_on_sc=False),
    scratch_types=[
        pltpu.VMEM((2, NGROUP, GROUP), jnp.int32),
        pltpu.VMEM((2, CHUNK, EMBED), jnp.float32),
        pltpu.VMEM((MAXLEN, EMBED), jnp.float32),
        pltpu.SemaphoreType.DMA,
        pltpu.SemaphoreType.DMA,
        pltpu.SemaphoreType.DMA,
        pltpu.SemaphoreType.DMA,
    ],
)
def _embed_sc(
    x_hbm, tok_hbm, pos_hbm, out_hbm,
    idx_v, rows_v, pos_v, sem_g0, sem_g1, sem_w0, sem_w1,
):
    wid = lax.axis_index("s") * NC + lax.axis_index("c")
    sem_g = (sem_g0, sem_g1)
    sem_w = (sem_w0, sem_w1)
    pltpu.sync_copy(pos_hbm, pos_v)

    def fire_gathers(j, par):
        pltpu.sync_copy(x_hbm.at[j], idx_v.at[par])
        for g in range(NGROUP):
            pltpu.async_copy(
                tok_hbm.at[idx_v.at[par, g]],
                rows_v.at[par, pl.ds(g * GROUP, GROUP)],
                sem_g[par],
            )

    def drain_gathers(j, par):
        for g in range(NGROUP):
            pltpu.make_async_copy(
                tok_hbm.at[idx_v.at[par, g]],
                rows_v.at[par, pl.ds(g * GROUP, GROUP)],
                sem_g[par],
            ).wait()

    def wait_write(j, par):
        pltpu.make_async_copy(rows_v.at[par], out_hbm.at[j], sem_w[par]).wait()

    fire_gathers(wid * NCHUNK, 0)

    def pair_body(i, _):
        for par in range(2):
            c = 2 * i + par
            j = wid * NCHUNK + c

            @pl.when(c + 1 < NCHUNK)
            def _():
                @pl.when(c >= 1)
                def _():
                    wait_write(j - 1, 1 - par)

                fire_gathers(j + 1, 1 - par)

            drain_gathers(j, par)

            p0 = lax.rem(c * CHUNK, MAXLEN)

            def row_body(r, p):
                p = jnp.where(p >= MAXLEN, p - MAXLEN, p)
                for h in range(HALVES):
                    sl = pl.ds(h * LANES, LANES)
                    plsc.addupdate(rows_v.at[par, r, sl], pos_v[p, sl])
                return p + 1

            lax.fori_loop(0, CHUNK, row_body, p0, unroll=4)
            pltpu.async_copy(rows_v.at[par], out_hbm.at[j], sem_w[par])
        return 0

    lax.fori_loop(0, NCHUNK // 2, pair_body, 0)
    wait_write(wid * NCHUNK + NCHUNK - 2, 0)
    wait_write(wid * NCHUNK + NCHUNK - 1, 1)


def kernel(x, token_table, pos_table):
    xr = x.astype(jnp.int32).reshape(NW * NCHUNK, NGROUP, GROUP)
    out = _embed_sc(xr, token_table, pos_table)
    return out.reshape(BATCH, MAXLEN, EMBED)


# speedup vs baseline: 1.2850x; 1.0060x over previous
import functools

import jax
import jax.numpy as jnp
from jax import lax
from jax.experimental import pallas as pl
from jax.experimental.pallas import tpu as pltpu
from jax.experimental.pallas import tpu_sc as plsc

MAXLEN = 200
EMBED = 32
BATCH = 4096

NC = 2
NS = 16
NW = NC * NS
LANES = 16

TOTAL = BATCH * MAXLEN
PER_W = TOTAL // NW
CHUNK = 1280
GROUP = 128
NGROUP = CHUNK // GROUP
NCHUNK = PER_W // CHUNK
HALVES = EMBED // LANES

_mesh = plsc.VectorSubcoreMesh(core_axis_name="c", subcore_axis_name="s")


@functools.partial(
    pl.kernel,
    out_type=jax.ShapeDtypeStruct((NW * NCHUNK, CHUNK, EMBED), jnp.float32),
    mesh=_mesh,
    compiler_params=pltpu.CompilerParams(use_tc_tiling_on_sc=False),
    scratch_types=[
        pltpu.VMEM((2, NGROUP, GROUP), jnp.int32),
        pltpu.VMEM((2, CHUNK, EMBED), jnp.float32),
        pltpu.VMEM((MAXLEN, EMBED), jnp.float32),
        pltpu.SemaphoreType.DMA,
        pltpu.SemaphoreType.DMA,
        pltpu.SemaphoreType.DMA,
        pltpu.SemaphoreType.DMA,
    ],
)
def _embed_sc(
    x_hbm, tok_hbm, pos_hbm, out_hbm,
    idx_v, rows_v, pos_v, sem_g0, sem_g1, sem_w0, sem_w1,
):
    wid = lax.axis_index("s") * NC + lax.axis_index("c")
    sem_g = (sem_g0, sem_g1)
    sem_w = (sem_w0, sem_w1)
    pltpu.sync_copy(pos_hbm, pos_v)

    def fire_gathers(j, par):
        pltpu.sync_copy(x_hbm.at[j], idx_v.at[par])
        for g in range(NGROUP):
            pltpu.async_copy(
                tok_hbm.at[idx_v.at[par, g]],
                rows_v.at[par, pl.ds(g * GROUP, GROUP)],
                sem_g[par],
            )

    def drain_gathers(j, par):
        for g in range(NGROUP):
            pltpu.make_async_copy(
                tok_hbm.at[idx_v.at[par, g]],
                rows_v.at[par, pl.ds(g * GROUP, GROUP)],
                sem_g[par],
            ).wait()

    def wait_write(j, par):
        pltpu.make_async_copy(rows_v.at[par], out_hbm.at[j], sem_w[par]).wait()

    fire_gathers(wid * NCHUNK, 0)

    def pair_body(i, _):
        for par in range(2):
            c = 2 * i + par
            j = wid * NCHUNK + c

            @pl.when(c + 1 < NCHUNK)
            def _():
                @pl.when(c >= 1)
                def _():
                    wait_write(j - 1, 1 - par)

                fire_gathers(j + 1, 1 - par)

            drain_gathers(j, par)

            p0 = lax.rem(c * CHUNK, MAXLEN)

            def row_body(r, p):
                p = jnp.where(p >= MAXLEN, p - MAXLEN, p)
                for h in range(HALVES):
                    sl = pl.ds(h * LANES, LANES)
                    plsc.addupdate(rows_v.at[par, r, sl], pos_v[p, sl])
                return p + 1

            lax.fori_loop(0, CHUNK, row_body, p0, unroll=4)
            pltpu.async_copy(rows_v.at[par], out_hbm.at[j], sem_w[par])
        return 0

    lax.fori_loop(0, NCHUNK // 2, pair_body, 0)
    wait_write(wid * NCHUNK + NCHUNK - 2, 0)
    wait_write(wid * NCHUNK + NCHUNK - 1, 1)


def kernel(x, token_table, pos_table):
    xr = x.astype(jnp.int32).reshape(NW * NCHUNK, NGROUP, GROUP)
    out = _embed_sc(xr, token_table, pos_table)
    return out.reshape(BATCH, MAXLEN, EMBED)


# speedup vs baseline: 1.2857x; 1.0006x over previous
import functools

import jax
import jax.numpy as jnp
from jax import lax
from jax.experimental import pallas as pl
from jax.experimental.pallas import tpu as pltpu
from jax.experimental.pallas import tpu_sc as plsc

MAXLEN = 200
EMBED = 32
BATCH = 4096

NC = 2
NS = 16
NW = NC * NS
LANES = 16

TOTAL = BATCH * MAXLEN
PER_W = TOTAL // NW
CHUNK = 1280
GROUP = 128
NGROUP = CHUNK // GROUP
NCHUNK = PER_W // CHUNK
HALVES = EMBED // LANES

_mesh = plsc.VectorSubcoreMesh(core_axis_name="c", subcore_axis_name="s")


@functools.partial(
    pl.kernel,
    out_type=jax.ShapeDtypeStruct((NW * NCHUNK, CHUNK, EMBED), jnp.float32),
    mesh=_mesh,
    compiler_params=pltpu.CompilerParams(use_tc_tiling_on_sc=False),
    scratch_types=[
        pltpu.VMEM((2, NGROUP, GROUP), jnp.int32),
        pltpu.VMEM((2, CHUNK, EMBED), jnp.float32),
        pltpu.VMEM((MAXLEN, EMBED), jnp.float32),
        pltpu.SemaphoreType.DMA,
        pltpu.SemaphoreType.DMA,
        pltpu.SemaphoreType.DMA,
        pltpu.SemaphoreType.DMA,
    ],
)
def _embed_sc(
    x_hbm, tok_hbm, pos_hbm, out_hbm,
    idx_v, rows_v, pos_v, sem_g0, sem_g1, sem_w0, sem_w1,
):
    wid = lax.axis_index("s") * NC + lax.axis_index("c")
    sem_g = (sem_g0, sem_g1)
    sem_w = (sem_w0, sem_w1)
    pltpu.sync_copy(pos_hbm, pos_v)

    def fire_gathers(j, par):
        pltpu.sync_copy(x_hbm.at[j], idx_v.at[par])
        for g in range(NGROUP):
            pltpu.async_copy(
                tok_hbm.at[idx_v.at[par, g]],
                rows_v.at[par, pl.ds(g * GROUP, GROUP)],
                sem_g[par],
            )

    def drain_gathers(j, par):
        for g in range(NGROUP):
            pltpu.make_async_copy(
                tok_hbm.at[idx_v.at[par, g]],
                rows_v.at[par, pl.ds(g * GROUP, GROUP)],
                sem_g[par],
            ).wait()

    def wait_write(j, par):
        pltpu.make_async_copy(rows_v.at[par], out_hbm.at[j], sem_w[par]).wait()

    fire_gathers(wid * NCHUNK, 0)

    def pair_body(i, _):
        for par in range(2):
            c = 2 * i + par
            j = wid * NCHUNK + c

            @pl.when(c + 1 < NCHUNK)
            def _():
                @pl.when(c >= 1)
                def _():
                    wait_write(j - 1, 1 - par)

                fire_gathers(j + 1, 1 - par)

            drain_gathers(j, par)

            p0 = lax.rem(c * CHUNK, MAXLEN)

            def row_body(r, p):
                p = jnp.where(p >= MAXLEN, p - MAXLEN, p)
                for h in range(HALVES):
                    sl = pl.ds(h * LANES, LANES)
                    plsc.addupdate(rows_v.at[par, r, sl], pos_v[p, sl])
                return p + 1

            lax.fori_loop(0, CHUNK, row_body, p0, unroll=8)
            pltpu.async_copy(rows_v.at[par], out_hbm.at[j], sem_w[par])
        return 0

    lax.fori_loop(0, NCHUNK // 2, pair_body, 0)
    wait_write(wid * NCHUNK + NCHUNK - 2, 0)
    wait_write(wid * NCHUNK + NCHUNK - 1, 1)


def kernel(x, token_table, pos_table):
    xr = x.astype(jnp.int32).reshape(NW * NCHUNK, NGROUP, GROUP)
    out = _embed_sc(xr, token_table, pos_table)
    return out.reshape(BATCH, MAXLEN, EMBED)


# speedup vs baseline: 1.2885x; 1.0021x over previous
import functools

import jax
import jax.numpy as jnp
from jax import lax
from jax.experimental import pallas as pl
from jax.experimental.pallas import tpu as pltpu
from jax.experimental.pallas import tpu_sc as plsc

MAXLEN = 200
EMBED = 32
BATCH = 4096

NC = 2
NS = 16
NW = NC * NS
LANES = 16

TOTAL = BATCH * MAXLEN
PER_W = TOTAL // NW
CHUNK = 1600
GROUP = 80
NGROUP = CHUNK // GROUP
NCHUNK = PER_W // CHUNK
HALVES = EMBED // LANES

_mesh = plsc.VectorSubcoreMesh(core_axis_name="c", subcore_axis_name="s")


@functools.partial(
    pl.kernel,
    out_type=jax.ShapeDtypeStruct((NW * NCHUNK, CHUNK, EMBED), jnp.float32),
    mesh=_mesh,
    compiler_params=pltpu.CompilerParams(use_tc_tiling_on_sc=False),
    scratch_types=[
        pltpu.VMEM((2, NGROUP, GROUP), jnp.int32),
        pltpu.VMEM((2, CHUNK, EMBED), jnp.float32),
        pltpu.VMEM((MAXLEN, EMBED), jnp.float32),
        pltpu.SemaphoreType.DMA,
        pltpu.SemaphoreType.DMA,
        pltpu.SemaphoreType.DMA,
        pltpu.SemaphoreType.DMA,
    ],
)
def _embed_sc(
    x_hbm, tok_hbm, pos_hbm, out_hbm,
    idx_v, rows_v, pos_v, sem_g0, sem_g1, sem_w0, sem_w1,
):
    wid = lax.axis_index("s") * NC + lax.axis_index("c")
    sem_g = (sem_g0, sem_g1)
    sem_w = (sem_w0, sem_w1)
    pltpu.sync_copy(pos_hbm, pos_v)

    def fire_gathers(j, par):
        pltpu.sync_copy(x_hbm.at[j], idx_v.at[par])
        for g in range(NGROUP):
            pltpu.async_copy(
                tok_hbm.at[idx_v.at[par, g]],
                rows_v.at[par, pl.ds(g * GROUP, GROUP)],
                sem_g[par],
            )

    def drain_gathers(j, par):
        for g in range(NGROUP):
            pltpu.make_async_copy(
                tok_hbm.at[idx_v.at[par, g]],
                rows_v.at[par, pl.ds(g * GROUP, GROUP)],
                sem_g[par],
            ).wait()

    def wait_write(j, par):
        pltpu.make_async_copy(rows_v.at[par], out_hbm.at[j], sem_w[par]).wait()

    fire_gathers(wid * NCHUNK, 0)

    def pair_body(i, _):
        for par in range(2):
            c = 2 * i + par
            j = wid * NCHUNK + c

            @pl.when(c + 1 < NCHUNK)
            def _():
                @pl.when(c >= 1)
                def _():
                    wait_write(j - 1, 1 - par)

                fire_gathers(j + 1, 1 - par)

            drain_gathers(j, par)

            p0 = lax.rem(c * CHUNK, MAXLEN)

            def row_body(r, p):
                p = jnp.where(p >= MAXLEN, p - MAXLEN, p)
                for h in range(HALVES):
                    sl = pl.ds(h * LANES, LANES)
                    plsc.addupdate(rows_v.at[par, r, sl], pos_v[p, sl])
                return p + 1

            lax.fori_loop(0, CHUNK, row_body, p0, unroll=8)
            pltpu.async_copy(rows_v.at[par], out_hbm.at[j], sem_w[par])
        return 0

    lax.fori_loop(0, NCHUNK // 2, pair_body, 0)
    wait_write(wid * NCHUNK + NCHUNK - 2, 0)
    wait_write(wid * NCHUNK + NCHUNK - 1, 1)


def kernel(x, token_table, pos_table):
    xr = x.astype(jnp.int32).reshape(NW * NCHUNK, NGROUP, GROUP)
    out = _embed_sc(xr, token_table, pos_table)
    return out.reshape(BATCH, MAXLEN, EMBED)
